# baseline (speedup 1.0000x reference)
import numpy as np
import jax
import jax.numpy as jnp
from jax.experimental import pallas as pl
from jax.experimental.pallas import tpu as pltpu

_N_MODES = (32, 32, 32)
_LMAX = 2
_RADIAL_BINS = 16
_EPS = 1e-06
_NH = 9


def _hp_sym_k(n):
    k = n // 2
    pos = np.arange(k + n % 2, dtype=np.float32)
    neg = np.arange(-k, 0, dtype=np.float32)
    return np.concatenate([pos, neg], axis=0)


def _hp_real_sph(coords, lmax, eps):
    x = coords[:, 0]
    y = coords[:, 1]
    z = coords[:, 2]
    r = np.maximum(np.linalg.norm(coords, axis=-1), eps)
    x = x / r
    y = y / r
    z = z / r
    basis = [0.28209479177387814 * np.ones_like(x)]
    if lmax >= 1:
        basis.extend([0.4886025119029199 * y, 0.4886025119029199 * z, 0.4886025119029199 * x])
    if lmax >= 2:
        basis.extend([
            1.0925484305920792 * x * y,
            1.0925484305920792 * y * z,
            0.31539156525252005 * (3.0 * z * z - 1.0),
            1.0925484305920792 * x * z,
            0.5462742152960396 * (x * x - y * y),
        ])
    basis = np.stack(basis, axis=-1)
    zero_mask = np.abs(coords).sum(axis=-1) < eps
    if zero_mask.any() and basis.shape[1] > 1:
        basis = basis.copy()
        basis[zero_mask, 1:] = 0.0
    return basis


def _hp_build_weights():
    kx = _hp_sym_k(_N_MODES[0])
    ky = _hp_sym_k(_N_MODES[1])
    kz = _hp_sym_k(_N_MODES[2])
    KX, KY, KZ = np.meshgrid(kx, ky, kz, indexing='ij')
    coords = np.stack([KX, KY, KZ], axis=-1).reshape(-1, 3)
    radii = np.linalg.norm(coords, axis=-1)
    max_r = max(float(radii.max()), 1.0)
    bin_edges = np.linspace(0.0, max_r + 1e-06, _RADIAL_BINS + 1)
    shell_ids = np.searchsorted(bin_edges[1:-1], radii, side='left')
    npts = coords.shape[0]
    w_fwd = np.zeros((npts, _RADIAL_BINS * _NH), dtype=np.float32)
    w_bwd = np.zeros((_RADIAL_BINS * _NH, npts), dtype=np.float32)
    for sid in range(_RADIAL_BINS):
        idx = np.nonzero(shell_ids == sid)[0]
        if idx.size == 0:
            continue
        basis = _hp_real_sph(coords[idx], _LMAX, _EPS).astype(np.float32)
        pinv = np.linalg.pinv(basis).astype(np.float32)
        w_fwd[idx, sid * _NH:(sid + 1) * _NH] = pinv.T
        w_bwd[sid * _NH:(sid + 1) * _NH, idx] = basis.T
    return w_fwd, w_bwd


_W_FWD_NP, _W_BWD_NP = _hp_build_weights()
_NPTS = _W_FWD_NP.shape[0]
_NCOEF = _W_FWD_NP.shape[1]

_PI = 8
_KT = _PI * 1024
_NIG = _N_MODES[0] // _PI
_NCJ = 2
_WF3_NP = _W_FWD_NP.reshape(_NIG, _KT, _NCOEF)
_WB3_NP = np.ascontiguousarray(
    _W_BWD_NP.reshape(_NCOEF, _NIG, _KT).transpose(1, 0, 2))


def _coeff_body(x_ref, w_ref, c_ref):
    @pl.when(pl.program_id(1) == 0)
    def _init():
        c_ref[...] = jnp.zeros_like(c_ref)

    rows = x_ref.shape[0] * x_ref.shape[1]
    xm = x_ref[...].reshape(rows, _KT)
    c_ref[0] += jnp.dot(xm, w_ref[0], preferred_element_type=jnp.float32)


def _recon_body(c_ref, w_ref, o_ref):
    rec = jnp.dot(c_ref[0], w_ref[0], preferred_element_type=jnp.float32)
    o_ref[...] = rec.reshape(o_ref.shape)


def kernel(x_fft_sliced):
    b, c, n0, n1, n2 = x_fft_sliced.shape
    cs = c // _NCJ
    rows = b * cs
    x4 = x_fft_sliced.reshape(b, c, n0, n1 * n2)
    wf = jnp.asarray(_WF3_NP)
    wb = jnp.asarray(_WB3_NP)

    coeff = pl.pallas_call(
        _coeff_body,
        grid=(_NCJ, _NIG),
        in_specs=[
            pl.BlockSpec((b, cs, _PI, n1 * n2), lambda j, k: (0, j, k, 0)),
            pl.BlockSpec((1, _KT, _NCOEF), lambda j, k: (k, 0, 0)),
        ],
        out_specs=pl.BlockSpec((1, rows, _NCOEF), lambda j, k: (j, 0, 0)),
        out_shape=jax.ShapeDtypeStruct((_NCJ, rows, _NCOEF), jnp.float32),
        compiler_params=pltpu.CompilerParams(
            dimension_semantics=("parallel", "arbitrary")),
    )(x4, wf)

    out = pl.pallas_call(
        _recon_body,
        grid=(_NCJ, _NIG),
        in_specs=[
            pl.BlockSpec((1, rows, _NCOEF), lambda j, k: (j, 0, 0)),
            pl.BlockSpec((1, _NCOEF, _KT), lambda j, k: (k, 0, 0)),
        ],
        out_specs=pl.BlockSpec((b, cs, _PI, n1 * n2), lambda j, k: (0, j, k, 0)),
        out_shape=jax.ShapeDtypeStruct((b, c, n0, n1 * n2), jnp.float32),
        compiler_params=pltpu.CompilerParams(
            dimension_semantics=("parallel", "parallel")),
    )(coeff, wb)

    return out.reshape(x_fft_sliced.shape)


# speedup vs baseline: 1.0627x; 1.0627x over previous
import numpy as np
import jax
import jax.numpy as jnp
from jax.experimental import pallas as pl
from jax.experimental.pallas import tpu as pltpu

_N_MODES = (32, 32, 32)
_LMAX = 2
_RADIAL_BINS = 16
_EPS = 1e-06
_NH = 9


def _hp_sym_k(n):
    k = n // 2
    pos = np.arange(k + n % 2, dtype=np.float32)
    neg = np.arange(-k, 0, dtype=np.float32)
    return np.concatenate([pos, neg], axis=0)


def _hp_real_sph(coords, lmax, eps):
    x = coords[:, 0]
    y = coords[:, 1]
    z = coords[:, 2]
    r = np.maximum(np.linalg.norm(coords, axis=-1), eps)
    x = x / r
    y = y / r
    z = z / r
    basis = [0.28209479177387814 * np.ones_like(x)]
    if lmax >= 1:
        basis.extend([0.4886025119029199 * y, 0.4886025119029199 * z, 0.4886025119029199 * x])
    if lmax >= 2:
        basis.extend([
            1.0925484305920792 * x * y,
            1.0925484305920792 * y * z,
            0.31539156525252005 * (3.0 * z * z - 1.0),
            1.0925484305920792 * x * z,
            0.5462742152960396 * (x * x - y * y),
        ])
    basis = np.stack(basis, axis=-1)
    zero_mask = np.abs(coords).sum(axis=-1) < eps
    if zero_mask.any() and basis.shape[1] > 1:
        basis = basis.copy()
        basis[zero_mask, 1:] = 0.0
    return basis


def _hp_build_weights():
    kx = _hp_sym_k(_N_MODES[0])
    ky = _hp_sym_k(_N_MODES[1])
    kz = _hp_sym_k(_N_MODES[2])
    KX, KY, KZ = np.meshgrid(kx, ky, kz, indexing='ij')
    coords = np.stack([KX, KY, KZ], axis=-1).reshape(-1, 3)
    radii = np.linalg.norm(coords, axis=-1)
    max_r = max(float(radii.max()), 1.0)
    bin_edges = np.linspace(0.0, max_r + 1e-06, _RADIAL_BINS + 1)
    shell_ids = np.searchsorted(bin_edges[1:-1], radii, side='left')
    npts = coords.shape[0]
    w_fwd = np.zeros((npts, _RADIAL_BINS * _NH), dtype=np.float32)
    w_bwd = np.zeros((_RADIAL_BINS * _NH, npts), dtype=np.float32)
    for sid in range(_RADIAL_BINS):
        idx = np.nonzero(shell_ids == sid)[0]
        if idx.size == 0:
            continue
        basis = _hp_real_sph(coords[idx], _LMAX, _EPS).astype(np.float32)
        pinv = np.linalg.pinv(basis).astype(np.float32)
        w_fwd[idx, sid * _NH:(sid + 1) * _NH] = pinv.T
        w_bwd[sid * _NH:(sid + 1) * _NH, idx] = basis.T
    return w_fwd, w_bwd


_W_FWD_NP, _W_BWD_NP = _hp_build_weights()
_NPTS = _W_FWD_NP.shape[0]
_NCOEF = _W_FWD_NP.shape[1]

_PI = 8
_KT = _PI * 1024
_NIG = _N_MODES[0] // _PI
_NCJ = 2
_WF3_NP = _W_FWD_NP.reshape(_NIG, _KT, _NCOEF)
_WB3_NP = np.ascontiguousarray(
    _W_BWD_NP.reshape(_NCOEF, _NIG, _KT).transpose(1, 0, 2))


def _coeff_body(x_ref, w_ref, c_ref):
    @pl.when(pl.program_id(1) == 0)
    def _init():
        c_ref[...] = jnp.zeros_like(c_ref)

    rows = x_ref.shape[0] * x_ref.shape[1]
    xm = x_ref[...].reshape(rows, _KT)
    c_ref[0] += jnp.dot(xm, w_ref[0].astype(jnp.float32),
                        preferred_element_type=jnp.float32)


def _recon_body(c_ref, w_ref, o_ref):
    rec = jnp.dot(c_ref[0], w_ref[0].astype(jnp.float32),
                  preferred_element_type=jnp.float32)
    o_ref[...] = rec.reshape(o_ref.shape)


def kernel(x_fft_sliced):
    b, c, n0, n1, n2 = x_fft_sliced.shape
    cs = c // _NCJ
    rows = b * cs
    x4 = x_fft_sliced.reshape(b, c, n0, n1 * n2)
    wf = jnp.asarray(_WF3_NP).astype(jnp.bfloat16)
    wb = jnp.asarray(_WB3_NP).astype(jnp.bfloat16)

    coeff = pl.pallas_call(
        _coeff_body,
        grid=(_NCJ, _NIG),
        in_specs=[
            pl.BlockSpec((b, cs, _PI, n1 * n2), lambda j, k: (0, j, k, 0)),
            pl.BlockSpec((1, _KT, _NCOEF), lambda j, k: (k, 0, 0)),
        ],
        out_specs=pl.BlockSpec((1, rows, _NCOEF), lambda j, k: (j, 0, 0)),
        out_shape=jax.ShapeDtypeStruct((_NCJ, rows, _NCOEF), jnp.float32),
        compiler_params=pltpu.CompilerParams(
            dimension_semantics=("parallel", "arbitrary")),
    )(x4, wf)

    out = pl.pallas_call(
        _recon_body,
        grid=(_NCJ, _NIG),
        in_specs=[
            pl.BlockSpec((1, rows, _NCOEF), lambda j, k: (j, 0, 0)),
            pl.BlockSpec((1, _NCOEF, _KT), lambda j, k: (k, 0, 0)),
        ],
        out_specs=pl.BlockSpec((b, cs, _PI, n1 * n2), lambda j, k: (0, j, k, 0)),
        out_shape=jax.ShapeDtypeStruct((b, c, n0, n1 * n2), jnp.float32),
        compiler_params=pltpu.CompilerParams(
            dimension_semantics=("parallel", "parallel")),
    )(coeff, wb)

    return out.reshape(x_fft_sliced.shape)
